# baseline (speedup 1.0000x reference)
import functools

import jax
import jax.numpy as jnp
from jax.experimental import pallas as pl
from jax.experimental.pallas import tpu as pltpu

DIM = 768
HIDDEN = 1024
NUM_EXPERTS = 64
NUM_ROUTED = 63
TOP_K = 2
CAPACITY = 192
S = 2048


def _router_kernel(x_ref, wr_ref, bias_ref, m_ref, r_ref, c_ref):
    x = x_ref[...]
    wr = wr_ref[...]
    bias = bias_ref[...]

    logits = jax.lax.dot_general(
        x, wr, dimension_numbers=(((1,), (1,)), ((), ())),
        preferred_element_type=jnp.float32)
    probs = jax.nn.sigmoid(logits * bias)
    col = jax.lax.broadcasted_iota(jnp.int32, (S, NUM_EXPERTS), 1)
    probs = jnp.where(col < NUM_ROUTED, probs, -1.0)

    m1 = jnp.max(probs, axis=1, keepdims=True)
    a1 = jnp.min(jnp.where(probs == m1, col, NUM_EXPERTS), axis=1,
                 keepdims=True)
    probs2 = jnp.where(col == a1, -2.0, probs)
    m2 = jnp.max(probs2, axis=1, keepdims=True)
    a2 = jnp.min(jnp.where(probs2 == m2, col, NUM_EXPERTS), axis=1,
                 keepdims=True)
    s = m1 + m2
    w_full = jnp.where(col == a1, m1 / s, 0.0) + jnp.where(col == a2, m2 / s,
                                                           0.0)

    bits = jax.lax.bitcast_convert_type(w_full, jnp.int32)
    lo = jnp.zeros((1, NUM_EXPERTS), jnp.int32)
    hi = jnp.full((1, NUM_EXPERTS), 0x3F800000, jnp.int32)

    def body(_, carry):
        lo, hi = carry
        mid = lo + (hi - lo) // 2
        cnt = jnp.sum((bits >= mid).astype(jnp.int32), axis=0,
                      keepdims=True)
        ge = cnt >= CAPACITY
        return jnp.where(ge, mid, lo), jnp.where(ge, hi, mid)

    lo, hi = jax.lax.fori_loop(0, 31, body, (lo, hi))
    sel = (bits >= lo) & (w_full > 0.0)

    ti = jax.lax.broadcasted_iota(jnp.int32, (S, S), 0)
    tj = jax.lax.broadcasted_iota(jnp.int32, (S, S), 1)
    ltri = (tj <= ti).astype(jnp.float32)
    sels = sel.astype(jnp.float32)
    rank = jax.lax.dot_general(
        ltri, sels, dimension_numbers=(((1,), (0,)), ((), ())),
        preferred_element_type=jnp.float32)
    rank = jnp.where(sel, rank, 0.0)

    m_ref[...] = jnp.where(sel, w_full, 0.0).T
    r_ref[...] = rank.T
    c_ref[...] = jnp.minimum(
        jnp.sum(sel.astype(jnp.int32), axis=0, keepdims=True), CAPACITY)


CHUNK = 64
NCHUNK = CAPACITY // CHUNK


def _expert_kernel(cnt_ref, x_ref, wg_ref, wu_ref, wd_ref,
                   wgs_ref, wus_ref, wds_ref, m_ref, r_ref, out_ref):
    e = pl.program_id(0)
    x = x_ref[...]

    @pl.when(e == 0)
    def _():
        wg = wgs_ref[...].astype(jnp.bfloat16)
        wu = wus_ref[...].astype(jnp.bfloat16)
        wd = wds_ref[...].astype(jnp.bfloat16)
        g = jax.lax.dot_general(
            x, wg, dimension_numbers=(((1,), (1,)), ((), ())),
            preferred_element_type=jnp.float32)
        u = jax.lax.dot_general(
            x, wu, dimension_numbers=(((1,), (1,)), ((), ())),
            preferred_element_type=jnp.float32)
        h = (jax.nn.silu(g) * u).astype(jnp.bfloat16)
        out_ref[...] = jax.lax.dot_general(
            h, wd, dimension_numbers=(((1,), (1,)), ((), ())),
            preferred_element_type=jnp.float32)

    @pl.when(e > 0)
    def _():
        wg = wg_ref[0].astype(jnp.bfloat16)
        wu = wu_ref[0].astype(jnp.bfloat16)
        wd = wd_ref[0].astype(jnp.bfloat16)
        m = m_ref[0, 0]
        rank = r_ref[0, 0]
        ranki = rank.astype(jnp.int32)
        cnt = cnt_ref[e - 1]

        for c in range(NCHUNK):
            @pl.when(cnt > c * CHUNK)
            def _(c=c):
                si = (jax.lax.broadcasted_iota(jnp.int32, (CHUNK, S), 0)
                      + (c * CHUNK + 1))
                onehot = ranki[None, :] == si
                a = onehot.astype(jnp.bfloat16)
                xg = jax.lax.dot_general(
                    a, x, dimension_numbers=(((1,), (0,)), ((), ())),
                    preferred_element_type=jnp.float32)
                xg = xg.astype(jnp.bfloat16)
                g = jax.lax.dot_general(
                    xg, wg, dimension_numbers=(((1,), (1,)), ((), ())),
                    preferred_element_type=jnp.float32)
                u = jax.lax.dot_general(
                    xg, wu, dimension_numbers=(((1,), (1,)), ((), ())),
                    preferred_element_type=jnp.float32)
                h = (jax.nn.silu(g) * u).astype(jnp.bfloat16)
                o = jax.lax.dot_general(
                    h, wd, dimension_numbers=(((1,), (1,)), ((), ())),
                    preferred_element_type=jnp.float32)
                aw = jnp.where(onehot, m[None, :], 0.0)
                out_ref[...] += jax.lax.dot_general(
                    aw.astype(jnp.bfloat16), o.astype(jnp.bfloat16),
                    dimension_numbers=(((0,), (0,)), ((), ())),
                    preferred_element_type=jnp.float32)


@functools.partial(jax.jit, static_argnames=())
def kernel(x, Wg_s, Wu_s, Wd_s, Wg_r, Wu_r, Wd_r, W_router, routing_bias):
    Bc, Sc, d = x.shape
    xf = x.reshape(Sc, d)

    wr_p = jnp.concatenate([W_router, jnp.zeros((1, d), jnp.float32)], axis=0)
    bias_p = jnp.concatenate([routing_bias,
                              jnp.zeros((1,), jnp.float32)])[None, :]

    m, r, cnt = pl.pallas_call(
        _router_kernel,
        out_shape=(
            jax.ShapeDtypeStruct((NUM_EXPERTS, Sc), jnp.float32),
            jax.ShapeDtypeStruct((NUM_EXPERTS, Sc), jnp.float32),
            jax.ShapeDtypeStruct((1, NUM_EXPERTS), jnp.int32),
        ),
    )(xf, wr_p, bias_p)
    cnt = cnt.reshape(NUM_EXPERTS)
    m3 = m.reshape(NUM_EXPERTS, 1, Sc)
    r3 = r.reshape(NUM_EXPERTS, 1, Sc)

    x_bf = xf.astype(jnp.bfloat16)

    def _widx(e, cref):
        return (jnp.maximum(e - 1, 0), 0, 0)

    def _const(e, cref):
        return (0, 0)

    out = pl.pallas_call(
        _expert_kernel,
        grid_spec=pltpu.PrefetchScalarGridSpec(
            num_scalar_prefetch=1,
            grid=(NUM_EXPERTS,),
            in_specs=[
                pl.BlockSpec((Sc, d), _const),
                pl.BlockSpec((1, HIDDEN, d), _widx),
                pl.BlockSpec((1, HIDDEN, d), _widx),
                pl.BlockSpec((1, d, HIDDEN), _widx),
                pl.BlockSpec((HIDDEN, d), _const),
                pl.BlockSpec((HIDDEN, d), _const),
                pl.BlockSpec((d, HIDDEN), _const),
                pl.BlockSpec((1, 1, Sc), _widx),
                pl.BlockSpec((1, 1, Sc), _widx),
            ],
            out_specs=pl.BlockSpec((Sc, d), _const),
        ),
        out_shape=jax.ShapeDtypeStruct((Sc, d), jnp.float32),
        compiler_params=pltpu.CompilerParams(
            dimension_semantics=("arbitrary",),
        ),
    )(cnt, x_bf, Wg_r, Wu_r, Wd_r, Wg_s, Wu_s, Wd_s, m3, r3)

    return out.reshape(Bc, Sc, d)


# speedup vs baseline: 1.1750x; 1.1750x over previous
import functools

import jax
import jax.numpy as jnp
from jax.experimental import pallas as pl
from jax.experimental.pallas import tpu as pltpu

DIM = 768
HIDDEN = 1024
NUM_EXPERTS = 64
NUM_ROUTED = 63
TOP_K = 2
CAPACITY = 192
S = 2048


def _router_kernel(x_ref, wr_ref, bias_ref, m_ref, r_ref):
    x = x_ref[...]
    wr = wr_ref[...]
    bias = bias_ref[...]

    logits = jax.lax.dot_general(
        x, wr, dimension_numbers=(((1,), (1,)), ((), ())),
        preferred_element_type=jnp.float32)
    probs = jax.nn.sigmoid(logits * bias)
    col = jax.lax.broadcasted_iota(jnp.int32, (S, NUM_EXPERTS), 1)
    probs = jnp.where(col < NUM_ROUTED, probs, -1.0)

    m1 = jnp.max(probs, axis=1, keepdims=True)
    a1 = jnp.min(jnp.where(probs == m1, col, NUM_EXPERTS), axis=1,
                 keepdims=True)
    probs2 = jnp.where(col == a1, -2.0, probs)
    m2 = jnp.max(probs2, axis=1, keepdims=True)
    a2 = jnp.min(jnp.where(probs2 == m2, col, NUM_EXPERTS), axis=1,
                 keepdims=True)
    s = m1 + m2
    w_full = jnp.where(col == a1, m1 / s, 0.0) + jnp.where(col == a2, m2 / s,
                                                           0.0)

    bits = jax.lax.bitcast_convert_type(w_full, jnp.int32)
    lo = jnp.zeros((1, NUM_EXPERTS), jnp.int32)
    hi = jnp.full((1, NUM_EXPERTS), 0x3F800000, jnp.int32)

    def body(_, carry):
        lo, hi = carry
        mid = lo + (hi - lo) // 2
        cnt = jnp.sum((bits >= mid).astype(jnp.int32), axis=0,
                      keepdims=True)
        ge = cnt >= CAPACITY
        return jnp.where(ge, mid, lo), jnp.where(ge, hi, mid)

    lo, hi = jax.lax.fori_loop(0, 31, body, (lo, hi))
    sel = (bits >= lo) & (w_full > 0.0)

    ti = jax.lax.broadcasted_iota(jnp.int32, (S, S), 0)
    tj = jax.lax.broadcasted_iota(jnp.int32, (S, S), 1)
    ltri = (tj <= ti).astype(jnp.float32)
    sels = sel.astype(jnp.float32)
    rank = jax.lax.dot_general(
        ltri, sels, dimension_numbers=(((1,), (0,)), ((), ())),
        preferred_element_type=jnp.float32)
    rank = jnp.where(sel, rank, 0.0)

    m_ref[...] = jnp.where(sel, w_full, 0.0).T
    r_ref[...] = rank.T


NSLICE = 8
HSL = HIDDEN // NSLICE


def _expert_kernel(x_ref, wg_ref, wu_ref, wd_ref, wgs_ref, wus_ref, wds_ref,
                   m_ref, r_ref, out_ref):
    e = pl.program_id(0)
    x = x_ref[...]

    wg = wg_ref[0].astype(jnp.bfloat16)
    wu = wu_ref[0].astype(jnp.bfloat16)
    wd = wd_ref[0].astype(jnp.bfloat16)
    m = m_ref[0, 0]
    rank = r_ref[0, 0]
    si = jax.lax.broadcasted_iota(jnp.int32, (CAPACITY, S), 0) + 1
    ranki = rank.astype(jnp.int32)
    onehot = ranki[None, :] == si
    a = onehot.astype(jnp.bfloat16)
    xg = jax.lax.dot_general(
        a, x, dimension_numbers=(((1,), (0,)), ((), ())),
        preferred_element_type=jnp.float32)
    xg = xg.astype(jnp.bfloat16)
    g = jax.lax.dot_general(
        xg, wg, dimension_numbers=(((1,), (1,)), ((), ())),
        preferred_element_type=jnp.float32)
    u = jax.lax.dot_general(
        xg, wu, dimension_numbers=(((1,), (1,)), ((), ())),
        preferred_element_type=jnp.float32)
    h = (jax.nn.silu(g) * u).astype(jnp.bfloat16)
    o = jax.lax.dot_general(
        h, wd, dimension_numbers=(((1,), (1,)), ((), ())),
        preferred_element_type=jnp.float32)
    aw = jnp.where(onehot, m[None, :], 0.0)
    scat = jax.lax.dot_general(
        aw.astype(jnp.bfloat16), o.astype(jnp.bfloat16),
        dimension_numbers=(((0,), (0,)), ((), ())),
        preferred_element_type=jnp.float32)

    @pl.when(e == 0)
    def _():
        out_ref[...] = scat

    @pl.when(e > 0)
    def _():
        out_ref[...] += scat

    @pl.when(e % NSLICE == 0)
    def _():
        wgs = wgs_ref[...].astype(jnp.bfloat16)
        wus = wus_ref[...].astype(jnp.bfloat16)
        wds = wds_ref[...].astype(jnp.bfloat16)
        gs = jax.lax.dot_general(
            x, wgs, dimension_numbers=(((1,), (1,)), ((), ())),
            preferred_element_type=jnp.float32)
        us = jax.lax.dot_general(
            x, wus, dimension_numbers=(((1,), (1,)), ((), ())),
            preferred_element_type=jnp.float32)
        hs = (jax.nn.silu(gs) * us).astype(jnp.bfloat16)
        out_ref[...] += jax.lax.dot_general(
            hs, wds, dimension_numbers=(((1,), (1,)), ((), ())),
            preferred_element_type=jnp.float32)


@functools.partial(jax.jit, static_argnames=())
def kernel(x, Wg_s, Wu_s, Wd_s, Wg_r, Wu_r, Wd_r, W_router, routing_bias):
    Bc, Sc, d = x.shape
    xf = x.reshape(Sc, d)

    wr_p = jnp.concatenate([W_router, jnp.zeros((1, d), jnp.float32)], axis=0)
    bias_p = jnp.concatenate([routing_bias,
                              jnp.zeros((1,), jnp.float32)])[None, :]

    m, r = pl.pallas_call(
        _router_kernel,
        out_shape=(
            jax.ShapeDtypeStruct((NUM_EXPERTS, Sc), jnp.float32),
            jax.ShapeDtypeStruct((NUM_EXPERTS, Sc), jnp.float32),
        ),
    )(xf, wr_p, bias_p)
    m3 = m.reshape(NUM_EXPERTS, 1, Sc)
    r3 = r.reshape(NUM_EXPERTS, 1, Sc)

    x_bf = xf.astype(jnp.bfloat16)

    out = pl.pallas_call(
        _expert_kernel,
        grid=(NUM_ROUTED,),
        in_specs=[
            pl.BlockSpec((Sc, d), lambda e: (0, 0)),
            pl.BlockSpec((1, HIDDEN, d), lambda e: (e, 0, 0)),
            pl.BlockSpec((1, HIDDEN, d), lambda e: (e, 0, 0)),
            pl.BlockSpec((1, d, HIDDEN), lambda e: (e, 0, 0)),
            pl.BlockSpec((HSL, d), lambda e: (e // NSLICE, 0)),
            pl.BlockSpec((HSL, d), lambda e: (e // NSLICE, 0)),
            pl.BlockSpec((d, HSL), lambda e: (0, e // NSLICE)),
            pl.BlockSpec((1, 1, Sc), lambda e: (e, 0, 0)),
            pl.BlockSpec((1, 1, Sc), lambda e: (e, 0, 0)),
        ],
        out_specs=pl.BlockSpec((Sc, d), lambda e: (0, 0)),
        out_shape=jax.ShapeDtypeStruct((Sc, d), jnp.float32),
        compiler_params=pltpu.CompilerParams(
            dimension_semantics=("arbitrary",),
        ),
    )(x_bf, Wg_r, Wu_r, Wd_r, Wg_s, Wu_s, Wd_s, m3, r3)

    return out.reshape(Bc, Sc, d)


# speedup vs baseline: 1.3119x; 1.1165x over previous
import functools

import jax
import jax.numpy as jnp
from jax.experimental import pallas as pl
from jax.experimental.pallas import tpu as pltpu

DIM = 768
HIDDEN = 1024
NUM_EXPERTS = 64
NUM_ROUTED = 63
TOP_K = 2
CAPACITY = 192
S = 2048


def _router_kernel(x_ref, wr_ref, bias_ref, m_ref, r_ref):
    x = x_ref[...]
    wr = wr_ref[...]
    bias = bias_ref[...]

    logits = jax.lax.dot_general(
        x, wr, dimension_numbers=(((1,), (1,)), ((), ())),
        preferred_element_type=jnp.float32)
    probs = jax.nn.sigmoid(logits * bias)
    col = jax.lax.broadcasted_iota(jnp.int32, (S, NUM_EXPERTS), 1)
    probs = jnp.where(col < NUM_ROUTED, probs, -1.0)

    m1 = jnp.max(probs, axis=1, keepdims=True)
    a1 = jnp.min(jnp.where(probs == m1, col, NUM_EXPERTS), axis=1,
                 keepdims=True)
    probs2 = jnp.where(col == a1, -2.0, probs)
    m2 = jnp.max(probs2, axis=1, keepdims=True)
    a2 = jnp.min(jnp.where(probs2 == m2, col, NUM_EXPERTS), axis=1,
                 keepdims=True)
    s = m1 + m2
    w_full = jnp.where(col == a1, m1 / s, 0.0) + jnp.where(col == a2, m2 / s,
                                                           0.0)

    bits = jax.lax.bitcast_convert_type(w_full, jnp.int32)
    lo = jnp.zeros((1, NUM_EXPERTS), jnp.int32)
    hi = jnp.full((1, NUM_EXPERTS), 0x3F800000, jnp.int32)

    def body(_, carry):
        lo, hi = carry
        mid = lo + (hi - lo) // 2
        cnt = jnp.sum((bits >= mid).astype(jnp.int32), axis=0,
                      keepdims=True)
        ge = cnt >= CAPACITY
        return jnp.where(ge, mid, lo), jnp.where(ge, hi, mid)

    lo, hi = jax.lax.fori_loop(0, 31, body, (lo, hi))
    sel = (bits >= lo) & (w_full > 0.0)

    ti = jax.lax.broadcasted_iota(jnp.int32, (S, S), 0)
    tj = jax.lax.broadcasted_iota(jnp.int32, (S, S), 1)
    ltri = (tj <= ti).astype(jnp.float32)
    sels = sel.astype(jnp.float32)
    rank = jax.lax.dot_general(
        ltri, sels, dimension_numbers=(((1,), (0,)), ((), ())),
        preferred_element_type=jnp.float32)
    rank = jnp.where(sel, rank, 0.0)

    m_ref[...] = jnp.where(sel, w_full, 0.0).T
    r_ref[...] = rank.T


def _expert_kernel(x_ref, wg_ref, wu_ref, wd_ref, wgs_ref, wus_ref, wds_ref,
                   m_ref, r_ref, out_ref):
    e = pl.program_id(0)
    x = x_ref[...]

    @pl.when(e == 0)
    def _():
        wg = wgs_ref[...].astype(jnp.bfloat16)
        wu = wus_ref[...].astype(jnp.bfloat16)
        wd = wds_ref[...].astype(jnp.bfloat16)
        g = jax.lax.dot_general(
            x, wg, dimension_numbers=(((1,), (1,)), ((), ())),
            preferred_element_type=jnp.float32)
        u = jax.lax.dot_general(
            x, wu, dimension_numbers=(((1,), (1,)), ((), ())),
            preferred_element_type=jnp.float32)
        h = (jax.nn.silu(g) * u).astype(jnp.bfloat16)
        out_ref[...] = jax.lax.dot_general(
            h, wd, dimension_numbers=(((1,), (1,)), ((), ())),
            preferred_element_type=jnp.float32)

    @pl.when(e > 0)
    def _():
        wg = wg_ref[0].astype(jnp.bfloat16)
        wu = wu_ref[0].astype(jnp.bfloat16)
        wd = wd_ref[0].astype(jnp.bfloat16)
        m = m_ref[0, 0]
        rank = r_ref[0, 0]
        si = jax.lax.broadcasted_iota(jnp.int32, (CAPACITY, S), 0) + 1
        ranki = rank.astype(jnp.int32)
        onehot = ranki[None, :] == si
        a = onehot.astype(jnp.bfloat16)
        xg = jax.lax.dot_general(
            a, x, dimension_numbers=(((1,), (0,)), ((), ())),
            preferred_element_type=jnp.float32)
        xg = xg.astype(jnp.bfloat16)
        g = jax.lax.dot_general(
            xg, wg, dimension_numbers=(((1,), (1,)), ((), ())),
            preferred_element_type=jnp.float32)
        u = jax.lax.dot_general(
            xg, wu, dimension_numbers=(((1,), (1,)), ((), ())),
            preferred_element_type=jnp.float32)
        h = (jax.nn.silu(g) * u).astype(jnp.bfloat16)
        o = jax.lax.dot_general(
            h, wd, dimension_numbers=(((1,), (1,)), ((), ())),
            preferred_element_type=jnp.float32)
        aw = jnp.where(onehot, m[None, :], 0.0)
        out_ref[...] += jax.lax.dot_general(
            aw.astype(jnp.bfloat16), o.astype(jnp.bfloat16),
            dimension_numbers=(((0,), (0,)), ((), ())),
            preferred_element_type=jnp.float32)


@functools.partial(jax.jit, static_argnames=())
def kernel(x, Wg_s, Wu_s, Wd_s, Wg_r, Wu_r, Wd_r, W_router, routing_bias):
    Bc, Sc, d = x.shape
    xf = x.reshape(Sc, d)

    wr_p = jnp.concatenate([W_router, jnp.zeros((1, d), jnp.float32)], axis=0)
    bias_p = jnp.concatenate([routing_bias,
                              jnp.zeros((1,), jnp.float32)])[None, :]

    m, r = pl.pallas_call(
        _router_kernel,
        out_shape=(
            jax.ShapeDtypeStruct((NUM_EXPERTS, Sc), jnp.float32),
            jax.ShapeDtypeStruct((NUM_EXPERTS, Sc), jnp.float32),
        ),
    )(xf, wr_p, bias_p)
    m3 = m.reshape(NUM_EXPERTS, 1, Sc)
    r3 = r.reshape(NUM_EXPERTS, 1, Sc)

    x_bf = xf.astype(jnp.bfloat16)

    def _widx(e):
        return (jnp.maximum(e - 1, 0), 0, 0)

    out = pl.pallas_call(
        _expert_kernel,
        grid=(NUM_EXPERTS,),
        in_specs=[
            pl.BlockSpec((Sc, d), lambda e: (0, 0)),
            pl.BlockSpec((1, HIDDEN, d), _widx),
            pl.BlockSpec((1, HIDDEN, d), _widx),
            pl.BlockSpec((1, d, HIDDEN), _widx),
            pl.BlockSpec((HIDDEN, d), lambda e: (0, 0)),
            pl.BlockSpec((HIDDEN, d), lambda e: (0, 0)),
            pl.BlockSpec((d, HIDDEN), lambda e: (0, 0)),
            pl.BlockSpec((1, 1, Sc), _widx),
            pl.BlockSpec((1, 1, Sc), _widx),
        ],
        out_specs=pl.BlockSpec((Sc, d), lambda e: (0, 0)),
        out_shape=jax.ShapeDtypeStruct((Sc, d), jnp.float32),
        compiler_params=pltpu.CompilerParams(
            dimension_semantics=("arbitrary",),
        ),
    )(x_bf, Wg_r, Wu_r, Wd_r, Wg_s, Wu_s, Wd_s, m3, r3)

    return out.reshape(Bc, Sc, d)
